# Initial kernel scaffold; baseline (speedup 1.0000x reference)
import functools

import jax
import jax.numpy as jnp
from jax import lax
from jax.experimental import pallas as pl
from jax.experimental.pallas import tpu as pltpu
from jax.experimental.pallas import tpu_sc as plsc

FEAT = 128
NC = 2
NS = 16
NW = NC * NS

SUB = 128
JCHUNK = 8
BIG = SUB * JCHUNK


def _round_up(x, m):
    return (x + m - 1) // m * m


@functools.lru_cache(maxsize=None)
def _make_segsum(n_nodes, nbig):
    n_acc = _round_up(n_nodes + 16, NS)
    zrows = n_acc // NS
    mesh = plsc.VectorSubcoreMesh(core_axis_name="c", subcore_axis_name="s")

    @functools.partial(
        pl.kernel,
        mesh=mesh,
        out_type=jax.ShapeDtypeStruct((NC, n_acc, FEAT), jnp.float32),
        scratch_types=[
            pltpu.VMEM((JCHUNK, SUB), jnp.int32),
            pltpu.VMEM((JCHUNK, SUB), jnp.int32),
            pltpu.VMEM((SUB, FEAT), jnp.float32),
            pltpu.VMEM_SHARED((10016, FEAT), jnp.float32),
            pltpu.SemaphoreType.DMA,
        ],
    )
    def segsum(sup_hbm, src_hbm, dst_hbm, zeros_hbm, out_hbm,
               src_v, dst_v, rows_v, accum, sem):
        c = lax.axis_index("c")
        s = lax.axis_index("s")
        wid = s * NC + c

        pltpu.sync_copy(zeros_hbm, accum.at[pl.ds(s * zrows, zrows)])
        plsc.subcore_barrier()

        def big_body(g, carry):
            pltpu.sync_copy(src_hbm.at[wid, g], src_v)
            pltpu.sync_copy(dst_hbm.at[wid, g], dst_v)
            for j in range(JCHUNK):
                pltpu.async_copy(sup_hbm.at[src_v.at[j]], rows_v, sem).wait()
                pltpu.sync_copy(rows_v, accum.at[dst_v.at[j]], add=True)
            return carry

        lax.fori_loop(0, nbig, big_body, 0)
        plsc.subcore_barrier()

        pltpu.sync_copy(accum.at[pl.ds(s * zrows, zrows)],
                        out_hbm.at[c, pl.ds(s * zrows, zrows)])

    return segsum


_ROWS_BLK = 2000


def _mm2_body(x_ref, w_ref, wl_ref, b_ref, sup_ref, lp_ref):
    xb = x_ref[...]
    sup_ref[...] = jnp.dot(xb, w_ref[...], preferred_element_type=jnp.float32)
    lp_ref[...] = (jnp.dot(xb, wl_ref[...], preferred_element_type=jnp.float32)
                   + b_ref[...])


def _tc_support_loop(x, w, wl, b):
    n = x.shape[0]
    grid = n // _ROWS_BLK
    return pl.pallas_call(
        _mm2_body,
        grid=(grid,),
        in_specs=[
            pl.BlockSpec((_ROWS_BLK, FEAT), lambda i: (i, 0)),
            pl.BlockSpec((FEAT, FEAT), lambda i: (0, 0)),
            pl.BlockSpec((FEAT, FEAT), lambda i: (0, 0)),
            pl.BlockSpec((1, FEAT), lambda i: (0, 0)),
        ],
        out_specs=[
            pl.BlockSpec((_ROWS_BLK, FEAT), lambda i: (i, 0)),
            pl.BlockSpec((_ROWS_BLK, FEAT), lambda i: (i, 0)),
        ],
        out_shape=[
            jax.ShapeDtypeStruct((n, FEAT), jnp.float32),
            jax.ShapeDtypeStruct((n, FEAT), jnp.float32),
        ],
    )(x, w, wl, b.reshape(1, FEAT))


def _relu_mm2_body(p_ref, lp_ref, w_ref, wl_ref, b_ref, sup_ref, lp2_ref):
    h = jnp.maximum(p_ref[0] + p_ref[1] + lp_ref[...], 0.0)
    sup_ref[...] = jnp.dot(h, w_ref[...], preferred_element_type=jnp.float32)
    lp2_ref[...] = (jnp.dot(h, wl_ref[...], preferred_element_type=jnp.float32)
                    + b_ref[...])


def _tc_combine_mm(p, lp, w, wl, b):
    n = lp.shape[0]
    grid = n // _ROWS_BLK
    return pl.pallas_call(
        _relu_mm2_body,
        grid=(grid,),
        in_specs=[
            pl.BlockSpec((2, _ROWS_BLK, FEAT), lambda i: (0, i, 0)),
            pl.BlockSpec((_ROWS_BLK, FEAT), lambda i: (i, 0)),
            pl.BlockSpec((FEAT, FEAT), lambda i: (0, 0)),
            pl.BlockSpec((FEAT, FEAT), lambda i: (0, 0)),
            pl.BlockSpec((1, FEAT), lambda i: (0, 0)),
        ],
        out_specs=[
            pl.BlockSpec((_ROWS_BLK, FEAT), lambda i: (i, 0)),
            pl.BlockSpec((_ROWS_BLK, FEAT), lambda i: (i, 0)),
        ],
        out_shape=[
            jax.ShapeDtypeStruct((n, FEAT), jnp.float32),
            jax.ShapeDtypeStruct((n, FEAT), jnp.float32),
        ],
    )(p, lp, w, wl, b.reshape(1, FEAT))


def _final_body(x_ref, p_ref, lp_ref, o_ref):
    h = jnp.maximum(p_ref[0] + p_ref[1] + lp_ref[...], 0.0)
    o_ref[...] = (x_ref[...] + h) * 0.5


def _tc_final(x, p, lp):
    n = x.shape[0]
    grid = n // _ROWS_BLK
    return pl.pallas_call(
        _final_body,
        grid=(grid,),
        in_specs=[
            pl.BlockSpec((_ROWS_BLK, FEAT), lambda i: (i, 0)),
            pl.BlockSpec((2, _ROWS_BLK, FEAT), lambda i: (0, i, 0)),
            pl.BlockSpec((_ROWS_BLK, FEAT), lambda i: (i, 0)),
        ],
        out_specs=pl.BlockSpec((_ROWS_BLK, FEAT), lambda i: (i, 0)),
        out_shape=jax.ShapeDtypeStruct((n, FEAT), jnp.float32),
    )(x, p, lp)


def kernel(inputs, adj_mat, W1, W1_loop, b1, W2, W2_loop, b2):
    n = inputs.shape[0]
    e = adj_mat.shape[1]
    nbig = -(-e // (NW * BIG))
    epad = NW * BIG * nbig - e
    zrows = _round_up(n + 16, NS) // NS

    src = adj_mat[0]
    dst = adj_mat[1]
    if epad:
        src = jnp.concatenate([src, jnp.zeros((epad,), jnp.int32)])
        dst = jnp.concatenate(
            [dst, n + (jnp.arange(epad, dtype=jnp.int32) % 16)])
    src_r = src.reshape(NW, nbig, JCHUNK, SUB)
    dst_r = dst.reshape(NW, nbig, JCHUNK, SUB)
    zeros = jnp.zeros((zrows, FEAT), jnp.float32)

    segsum = _make_segsum(n, nbig)

    sup1, lp1 = _tc_support_loop(inputs, W1, W1_loop, b1)
    p1 = segsum(sup1, src_r, dst_r, zeros)
    sup2, lp2 = _tc_combine_mm(p1, lp1, W2, W2_loop, b2)
    p2 = segsum(sup2, src_r, dst_r, zeros)
    return _tc_final(inputs, p2, lp2)


# speedup vs baseline: 2.8497x; 2.8497x over previous
import functools

import jax
import jax.numpy as jnp
from jax import lax
from jax.experimental import pallas as pl
from jax.experimental.pallas import tpu as pltpu
from jax.experimental.pallas import tpu_sc as plsc

FEAT = 128
NC = 2
NS = 16
NW = NC * NS

SUB = 128
JCHUNK = 8
BIG = SUB * JCHUNK


def _round_up(x, m):
    return (x + m - 1) // m * m


@functools.lru_cache(maxsize=None)
def _make_segsum(n_nodes, nbig):
    n_acc = _round_up(n_nodes + 16, NS * 8)
    zrows = n_acc // NS
    mesh = plsc.VectorSubcoreMesh(core_axis_name="c", subcore_axis_name="s")

    @functools.partial(
        pl.kernel,
        mesh=mesh,
        out_type=jax.ShapeDtypeStruct((NC, n_acc, FEAT), jnp.float32),
        scratch_types=[
            pltpu.VMEM((JCHUNK, SUB), jnp.int32),
            pltpu.VMEM((JCHUNK, SUB), jnp.int32),
            pltpu.VMEM((SUB, FEAT), jnp.float32),
            pltpu.VMEM_SHARED((n_acc, FEAT), jnp.float32),
            pltpu.SemaphoreType.DMA,
        ],
    )
    def segsum(sup_hbm, src_hbm, dst_hbm, zeros_hbm, out_hbm,
               src_v, dst_v, rows_v, accum, sem):
        c = lax.axis_index("c")
        s = lax.axis_index("s")
        wid = s * NC + c

        pltpu.sync_copy(zeros_hbm, accum.at[pl.ds(s * zrows, zrows)])
        plsc.subcore_barrier()

        def big_body(g, carry):
            pltpu.sync_copy(src_hbm.at[wid, g], src_v)
            pltpu.sync_copy(dst_hbm.at[wid, g], dst_v)
            for j in range(JCHUNK):
                pltpu.async_copy(sup_hbm.at[src_v.at[j]], rows_v, sem).wait()
                pltpu.sync_copy(rows_v, accum.at[dst_v.at[j]], add=True)
            return carry

        lax.fori_loop(0, nbig, big_body, 0)
        plsc.subcore_barrier()

        pltpu.sync_copy(accum.at[pl.ds(s * zrows, zrows)],
                        out_hbm.at[c, pl.ds(s * zrows, zrows)])

    return segsum


_ROWS_BLK = 2000


def _mm2_body(x_ref, w_ref, wl_ref, b_ref, sup_ref, lp_ref):
    xb = x_ref[...]
    sup_ref[...] = jnp.dot(xb, w_ref[...], preferred_element_type=jnp.float32)
    lp_ref[...] = (jnp.dot(xb, wl_ref[...], preferred_element_type=jnp.float32)
                   + b_ref[...])


def _tc_support_loop(x, w, wl, b):
    n = x.shape[0]
    grid = n // _ROWS_BLK
    return pl.pallas_call(
        _mm2_body,
        grid=(grid,),
        in_specs=[
            pl.BlockSpec((_ROWS_BLK, FEAT), lambda i: (i, 0)),
            pl.BlockSpec((FEAT, FEAT), lambda i: (0, 0)),
            pl.BlockSpec((FEAT, FEAT), lambda i: (0, 0)),
            pl.BlockSpec((1, FEAT), lambda i: (0, 0)),
        ],
        out_specs=[
            pl.BlockSpec((_ROWS_BLK, FEAT), lambda i: (i, 0)),
            pl.BlockSpec((_ROWS_BLK, FEAT), lambda i: (i, 0)),
        ],
        out_shape=[
            jax.ShapeDtypeStruct((n, FEAT), jnp.float32),
            jax.ShapeDtypeStruct((n, FEAT), jnp.float32),
        ],
    )(x, w, wl, b.reshape(1, FEAT))


def _relu_mm2_body(p_ref, lp_ref, w_ref, wl_ref, b_ref, sup_ref, lp2_ref):
    h = jnp.maximum(p_ref[0] + p_ref[1] + lp_ref[...], 0.0)
    sup_ref[...] = jnp.dot(h, w_ref[...], preferred_element_type=jnp.float32)
    lp2_ref[...] = (jnp.dot(h, wl_ref[...], preferred_element_type=jnp.float32)
                    + b_ref[...])


def _tc_combine_mm(p, lp, w, wl, b):
    n = lp.shape[0]
    grid = n // _ROWS_BLK
    return pl.pallas_call(
        _relu_mm2_body,
        grid=(grid,),
        in_specs=[
            pl.BlockSpec((2, _ROWS_BLK, FEAT), lambda i: (0, i, 0)),
            pl.BlockSpec((_ROWS_BLK, FEAT), lambda i: (i, 0)),
            pl.BlockSpec((FEAT, FEAT), lambda i: (0, 0)),
            pl.BlockSpec((FEAT, FEAT), lambda i: (0, 0)),
            pl.BlockSpec((1, FEAT), lambda i: (0, 0)),
        ],
        out_specs=[
            pl.BlockSpec((_ROWS_BLK, FEAT), lambda i: (i, 0)),
            pl.BlockSpec((_ROWS_BLK, FEAT), lambda i: (i, 0)),
        ],
        out_shape=[
            jax.ShapeDtypeStruct((n, FEAT), jnp.float32),
            jax.ShapeDtypeStruct((n, FEAT), jnp.float32),
        ],
    )(p, lp, w, wl, b.reshape(1, FEAT))


def _final_body(x_ref, p_ref, lp_ref, o_ref):
    h = jnp.maximum(p_ref[0] + p_ref[1] + lp_ref[...], 0.0)
    o_ref[...] = (x_ref[...] + h) * 0.5


def _tc_final(x, p, lp):
    n = x.shape[0]
    grid = n // _ROWS_BLK
    return pl.pallas_call(
        _final_body,
        grid=(grid,),
        in_specs=[
            pl.BlockSpec((_ROWS_BLK, FEAT), lambda i: (i, 0)),
            pl.BlockSpec((2, _ROWS_BLK, FEAT), lambda i: (0, i, 0)),
            pl.BlockSpec((_ROWS_BLK, FEAT), lambda i: (i, 0)),
        ],
        out_specs=pl.BlockSpec((_ROWS_BLK, FEAT), lambda i: (i, 0)),
        out_shape=jax.ShapeDtypeStruct((n, FEAT), jnp.float32),
    )(x, p, lp)


def kernel(inputs, adj_mat, W1, W1_loop, b1, W2, W2_loop, b2):
    n = inputs.shape[0]
    e = adj_mat.shape[1]
    nbig = -(-e // (NW * BIG))
    epad = NW * BIG * nbig - e
    zrows = _round_up(n + 16, NS * 8) // NS

    src = adj_mat[0]
    dst = adj_mat[1]
    if epad:
        src = jnp.concatenate([src, jnp.zeros((epad,), jnp.int32)])
        dst = jnp.concatenate(
            [dst, n + (jnp.arange(epad, dtype=jnp.int32) % 16)])
    src_r = src.reshape(NW, nbig, JCHUNK, SUB)
    dst_r = dst.reshape(NW, nbig, JCHUNK, SUB)
    zeros = jnp.zeros((zrows, FEAT), jnp.float32)

    segsum = _make_segsum(n, nbig)

    sup1, lp1 = _tc_support_loop(inputs, W1, W1_loop, b1)
    p1 = segsum(sup1, src_r, dst_r, zeros)
    sup2, lp2 = _tc_combine_mm(p1, lp1, W2, W2_loop, b2)
    p2 = segsum(sup2, src_r, dst_r, zeros)
    return _tc_final(inputs, p2, lp2)


# speedup vs baseline: 3.0124x; 1.0571x over previous
import functools

import jax
import jax.numpy as jnp
from jax import lax
from jax.experimental import pallas as pl
from jax.experimental.pallas import tpu as pltpu
from jax.experimental.pallas import tpu_sc as plsc

FEAT = 128
NC = 2
NS = 16
NW = NC * NS

SUB = 128
JCHUNK = 8
BIG = SUB * JCHUNK


def _round_up(x, m):
    return (x + m - 1) // m * m


@functools.lru_cache(maxsize=None)
def _make_segsum(n_nodes, nbig):
    n_acc = _round_up(n_nodes + 16, NS * 8)
    zrows = n_acc // NS
    mesh = plsc.VectorSubcoreMesh(core_axis_name="c", subcore_axis_name="s")

    @functools.partial(
        pl.kernel,
        mesh=mesh,
        out_type=jax.ShapeDtypeStruct((NC, n_acc, FEAT), jnp.float32),
        scratch_types=[
            pltpu.VMEM((JCHUNK, SUB), jnp.int32),
            pltpu.VMEM((JCHUNK, SUB), jnp.int32),
            pltpu.VMEM((SUB, FEAT), jnp.float32),
            pltpu.VMEM((SUB, FEAT), jnp.float32),
            pltpu.VMEM_SHARED((n_acc, FEAT), jnp.float32),
            pltpu.SemaphoreType.DMA,
            pltpu.SemaphoreType.DMA,
        ],
    )
    def segsum(sup_hbm, src_hbm, dst_hbm, zeros_hbm, out_hbm,
               src_v, dst_v, rows_v0, rows_v1, accum, gsem, ssem):
        c = lax.axis_index("c")
        s = lax.axis_index("s")
        wid = s * NC + c
        rows = (rows_v0, rows_v1)

        pltpu.sync_copy(zeros_hbm, accum.at[pl.ds(s * zrows, zrows)])
        plsc.subcore_barrier()

        def big_body(g, carry):
            pltpu.sync_copy(src_hbm.at[wid, g], src_v)
            pltpu.sync_copy(dst_hbm.at[wid, g], dst_v)
            gh = [None] * JCHUNK
            sh = [None] * JCHUNK
            gh[0] = pltpu.async_copy(sup_hbm.at[src_v.at[0]], rows[0], gsem)
            for j in range(JCHUNK):
                cur = rows[j % 2]
                nxt = rows[(j + 1) % 2]
                gh[j].wait()
                if j >= 1:
                    sh[j - 1].wait()
                if j + 1 < JCHUNK:
                    gh[j + 1] = pltpu.async_copy(
                        sup_hbm.at[src_v.at[j + 1]], nxt, gsem)
                sh[j] = pltpu.async_copy(
                    cur, accum.at[dst_v.at[j]], ssem, add=True)
            sh[JCHUNK - 1].wait()
            return carry

        lax.fori_loop(0, nbig, big_body, 0)
        plsc.subcore_barrier()

        pltpu.sync_copy(accum.at[pl.ds(s * zrows, zrows)],
                        out_hbm.at[c, pl.ds(s * zrows, zrows)])

    return segsum


_ROWS_BLK = 2000


def _mm2_body(x_ref, w_ref, wl_ref, b_ref, sup_ref, lp_ref):
    xb = x_ref[...]
    sup_ref[...] = jnp.dot(xb, w_ref[...], preferred_element_type=jnp.float32)
    lp_ref[...] = (jnp.dot(xb, wl_ref[...], preferred_element_type=jnp.float32)
                   + b_ref[...])


def _tc_support_loop(x, w, wl, b):
    n = x.shape[0]
    grid = n // _ROWS_BLK
    return pl.pallas_call(
        _mm2_body,
        grid=(grid,),
        in_specs=[
            pl.BlockSpec((_ROWS_BLK, FEAT), lambda i: (i, 0)),
            pl.BlockSpec((FEAT, FEAT), lambda i: (0, 0)),
            pl.BlockSpec((FEAT, FEAT), lambda i: (0, 0)),
            pl.BlockSpec((1, FEAT), lambda i: (0, 0)),
        ],
        out_specs=[
            pl.BlockSpec((_ROWS_BLK, FEAT), lambda i: (i, 0)),
            pl.BlockSpec((_ROWS_BLK, FEAT), lambda i: (i, 0)),
        ],
        out_shape=[
            jax.ShapeDtypeStruct((n, FEAT), jnp.float32),
            jax.ShapeDtypeStruct((n, FEAT), jnp.float32),
        ],
    )(x, w, wl, b.reshape(1, FEAT))


def _relu_mm2_body(p_ref, lp_ref, w_ref, wl_ref, b_ref, sup_ref, lp2_ref):
    h = jnp.maximum(p_ref[0] + p_ref[1] + lp_ref[...], 0.0)
    sup_ref[...] = jnp.dot(h, w_ref[...], preferred_element_type=jnp.float32)
    lp2_ref[...] = (jnp.dot(h, wl_ref[...], preferred_element_type=jnp.float32)
                    + b_ref[...])


def _tc_combine_mm(p, lp, w, wl, b):
    n = lp.shape[0]
    grid = n // _ROWS_BLK
    return pl.pallas_call(
        _relu_mm2_body,
        grid=(grid,),
        in_specs=[
            pl.BlockSpec((2, _ROWS_BLK, FEAT), lambda i: (0, i, 0)),
            pl.BlockSpec((_ROWS_BLK, FEAT), lambda i: (i, 0)),
            pl.BlockSpec((FEAT, FEAT), lambda i: (0, 0)),
            pl.BlockSpec((FEAT, FEAT), lambda i: (0, 0)),
            pl.BlockSpec((1, FEAT), lambda i: (0, 0)),
        ],
        out_specs=[
            pl.BlockSpec((_ROWS_BLK, FEAT), lambda i: (i, 0)),
            pl.BlockSpec((_ROWS_BLK, FEAT), lambda i: (i, 0)),
        ],
        out_shape=[
            jax.ShapeDtypeStruct((n, FEAT), jnp.float32),
            jax.ShapeDtypeStruct((n, FEAT), jnp.float32),
        ],
    )(p, lp, w, wl, b.reshape(1, FEAT))


def _final_body(x_ref, p_ref, lp_ref, o_ref):
    h = jnp.maximum(p_ref[0] + p_ref[1] + lp_ref[...], 0.0)
    o_ref[...] = (x_ref[...] + h) * 0.5


def _tc_final(x, p, lp):
    n = x.shape[0]
    grid = n // _ROWS_BLK
    return pl.pallas_call(
        _final_body,
        grid=(grid,),
        in_specs=[
            pl.BlockSpec((_ROWS_BLK, FEAT), lambda i: (i, 0)),
            pl.BlockSpec((2, _ROWS_BLK, FEAT), lambda i: (0, i, 0)),
            pl.BlockSpec((_ROWS_BLK, FEAT), lambda i: (i, 0)),
        ],
        out_specs=pl.BlockSpec((_ROWS_BLK, FEAT), lambda i: (i, 0)),
        out_shape=jax.ShapeDtypeStruct((n, FEAT), jnp.float32),
    )(x, p, lp)


def kernel(inputs, adj_mat, W1, W1_loop, b1, W2, W2_loop, b2):
    n = inputs.shape[0]
    e = adj_mat.shape[1]
    nbig = -(-e // (NW * BIG))
    epad = NW * BIG * nbig - e
    zrows = _round_up(n + 16, NS * 8) // NS

    src = adj_mat[0]
    dst = adj_mat[1]
    if epad:
        src = jnp.concatenate([src, jnp.zeros((epad,), jnp.int32)])
        dst = jnp.concatenate(
            [dst, n + (jnp.arange(epad, dtype=jnp.int32) % 16)])
    src_r = src.reshape(NW, nbig, JCHUNK, SUB)
    dst_r = dst.reshape(NW, nbig, JCHUNK, SUB)
    zeros = jnp.zeros((zrows, FEAT), jnp.float32)

    segsum = _make_segsum(n, nbig)

    sup1, lp1 = _tc_support_loop(inputs, W1, W1_loop, b1)
    p1 = segsum(sup1, src_r, dst_r, zeros)
    sup2, lp2 = _tc_combine_mm(p1, lp1, W2, W2_loop, b2)
    p2 = segsum(sup2, src_r, dst_r, zeros)
    return _tc_final(inputs, p2, lp2)


# speedup vs baseline: 8.9077x; 2.9570x over previous
import functools

import jax
import jax.numpy as jnp
from jax import lax
from jax.experimental import pallas as pl
from jax.experimental.pallas import tpu as pltpu
from jax.experimental.pallas import tpu_sc as plsc

FEAT = 128
NC = 2
NS = 16
NW = NC * NS

JCHUNK = 8


def _round_up(x, m):
    return (x + m - 1) // m * m


@functools.lru_cache(maxsize=None)
def _make_segsum(n_nodes, nbig, sub):
    n_acc = _round_up(n_nodes + 16, NS * 8)
    zrows = n_acc // NS
    mesh = plsc.VectorSubcoreMesh(core_axis_name="c", subcore_axis_name="s")

    @functools.partial(
        pl.kernel,
        mesh=mesh,
        out_type=jax.ShapeDtypeStruct((NC, n_acc, FEAT), jnp.float32),
        scratch_types=[
            pltpu.VMEM((JCHUNK, sub), jnp.int32),
            pltpu.VMEM((JCHUNK, sub), jnp.int32),
            pltpu.VMEM((sub, FEAT), jnp.float32),
            pltpu.VMEM((sub, FEAT), jnp.float32),
            pltpu.VMEM_SHARED((n_acc, FEAT), jnp.float32),
            pltpu.SemaphoreType.DMA,
            pltpu.SemaphoreType.DMA,
        ],
    )
    def segsum(sup_hbm, src_hbm, dst_hbm, zeros_hbm, out_hbm,
               src_v, dst_v, rows_v0, rows_v1, accum, gsem, ssem):
        c = lax.axis_index("c")
        s = lax.axis_index("s")
        wid = s * NC + c
        rows = (rows_v0, rows_v1)

        pltpu.sync_copy(zeros_hbm, accum.at[pl.ds(s * zrows, zrows)])
        plsc.subcore_barrier()

        def big_body(g, carry):
            pltpu.sync_copy(src_hbm.at[wid, g], src_v)
            pltpu.sync_copy(dst_hbm.at[wid, g], dst_v)
            gh = [None] * JCHUNK
            sh = [None] * JCHUNK
            gh[0] = pltpu.async_copy(sup_hbm.at[src_v.at[0]], rows[0], gsem)
            for j in range(JCHUNK):
                cur = rows[j % 2]
                nxt = rows[(j + 1) % 2]
                gh[j].wait()
                if j >= 1:
                    sh[j - 1].wait()
                if j + 1 < JCHUNK:
                    gh[j + 1] = pltpu.async_copy(
                        sup_hbm.at[src_v.at[j + 1]], nxt, gsem)
                sh[j] = pltpu.async_copy(
                    cur, accum.at[dst_v.at[j]], ssem, add=True)
            sh[JCHUNK - 1].wait()
            return carry

        lax.fori_loop(0, nbig, big_body, 0)
        plsc.subcore_barrier()

        pltpu.sync_copy(accum.at[pl.ds(s * zrows, zrows)],
                        out_hbm.at[c, pl.ds(s * zrows, zrows)])

    return segsum


_ROWS_BLK = 2000


def _mm2_body(x_ref, w_ref, wl_ref, b_ref, sup_ref, lp_ref):
    xb = x_ref[...]
    sup_ref[...] = jnp.dot(xb, w_ref[...], preferred_element_type=jnp.float32)
    lp_ref[...] = (jnp.dot(xb, wl_ref[...], preferred_element_type=jnp.float32)
                   + b_ref[...])


def _tc_support_loop(x, w, wl, b):
    n = x.shape[0]
    grid = n // _ROWS_BLK
    return pl.pallas_call(
        _mm2_body,
        grid=(grid,),
        in_specs=[
            pl.BlockSpec((_ROWS_BLK, FEAT), lambda i: (i, 0)),
            pl.BlockSpec((FEAT, FEAT), lambda i: (0, 0)),
            pl.BlockSpec((FEAT, FEAT), lambda i: (0, 0)),
            pl.BlockSpec((1, FEAT), lambda i: (0, 0)),
        ],
        out_specs=[
            pl.BlockSpec((_ROWS_BLK, FEAT), lambda i: (i, 0)),
            pl.BlockSpec((_ROWS_BLK, FEAT), lambda i: (i, 0)),
        ],
        out_shape=[
            jax.ShapeDtypeStruct((n, FEAT), jnp.float32),
            jax.ShapeDtypeStruct((n, FEAT), jnp.float32),
        ],
    )(x, w, wl, b.reshape(1, FEAT))


def _relu_mm2_body(p_ref, lp_ref, w_ref, wl_ref, b_ref, sup_ref, lp2_ref):
    h = jnp.maximum(p_ref[0] + p_ref[1] + lp_ref[...], 0.0)
    sup_ref[...] = jnp.dot(h, w_ref[...], preferred_element_type=jnp.float32)
    lp2_ref[...] = (jnp.dot(h, wl_ref[...], preferred_element_type=jnp.float32)
                    + b_ref[...])


def _tc_combine_mm(p, lp, w, wl, b):
    n = lp.shape[0]
    grid = n // _ROWS_BLK
    return pl.pallas_call(
        _relu_mm2_body,
        grid=(grid,),
        in_specs=[
            pl.BlockSpec((2, _ROWS_BLK, FEAT), lambda i: (0, i, 0)),
            pl.BlockSpec((_ROWS_BLK, FEAT), lambda i: (i, 0)),
            pl.BlockSpec((FEAT, FEAT), lambda i: (0, 0)),
            pl.BlockSpec((FEAT, FEAT), lambda i: (0, 0)),
            pl.BlockSpec((1, FEAT), lambda i: (0, 0)),
        ],
        out_specs=[
            pl.BlockSpec((_ROWS_BLK, FEAT), lambda i: (i, 0)),
            pl.BlockSpec((_ROWS_BLK, FEAT), lambda i: (i, 0)),
        ],
        out_shape=[
            jax.ShapeDtypeStruct((n, FEAT), jnp.float32),
            jax.ShapeDtypeStruct((n, FEAT), jnp.float32),
        ],
    )(p, lp, w, wl, b.reshape(1, FEAT))


def _final_body(x_ref, p_ref, lp_ref, o_ref):
    h = jnp.maximum(p_ref[0] + p_ref[1] + lp_ref[...], 0.0)
    o_ref[...] = (x_ref[...] + h) * 0.5


def _tc_final(x, p, lp):
    n = x.shape[0]
    grid = n // _ROWS_BLK
    return pl.pallas_call(
        _final_body,
        grid=(grid,),
        in_specs=[
            pl.BlockSpec((_ROWS_BLK, FEAT), lambda i: (i, 0)),
            pl.BlockSpec((2, _ROWS_BLK, FEAT), lambda i: (0, i, 0)),
            pl.BlockSpec((_ROWS_BLK, FEAT), lambda i: (i, 0)),
        ],
        out_specs=pl.BlockSpec((_ROWS_BLK, FEAT), lambda i: (i, 0)),
        out_shape=jax.ShapeDtypeStruct((n, FEAT), jnp.float32),
    )(x, p, lp)


def kernel(inputs, adj_mat, W1, W1_loop, b1, W2, W2_loop, b2):
    n = inputs.shape[0]
    e = adj_mat.shape[1]
    per_tile = -(-e // NW)
    nbig = -(-per_tile // (JCHUNK * 128))
    sub = -(-per_tile // (JCHUNK * nbig))
    epad = NW * nbig * JCHUNK * sub - e
    zrows = _round_up(n + 16, NS * 8) // NS

    src = adj_mat[0]
    dst = adj_mat[1]
    if epad:
        src = jnp.concatenate([src, jnp.zeros((epad,), jnp.int32)])
        dst = jnp.concatenate(
            [dst, n + (jnp.arange(epad, dtype=jnp.int32) % 96)])
    src_r = src.reshape(NW, nbig, JCHUNK, sub)
    dst_r = dst.reshape(NW, nbig, JCHUNK, sub)
    zeros = jnp.zeros((zrows, FEAT), jnp.float32)

    segsum = _make_segsum(n, nbig, sub)

    sup1, lp1 = _tc_support_loop(inputs, W1, W1_loop, b1)
    p1 = segsum(sup1, src_r, dst_r, zeros)
    sup2, lp2 = _tc_combine_mm(p1, lp1, W2, W2_loop, b2)
    p2 = segsum(sup2, src_r, dst_r, zeros)
    return _tc_final(inputs, p2, lp2)


# speedup vs baseline: 10.7936x; 1.2117x over previous
import functools

import jax
import jax.numpy as jnp
from jax import lax
from jax.experimental import pallas as pl
from jax.experimental.pallas import tpu as pltpu
from jax.experimental.pallas import tpu_sc as plsc

FEAT = 128
NC = 2
NS = 16
NW = NC * NS

JCHUNK = 10
SUB_CAP = 104


def _round_up(x, m):
    return (x + m - 1) // m * m


@functools.lru_cache(maxsize=None)
def _make_segsum(n_nodes, nbig, sub, n_acc):
    chunk = _round_up(-(-n_acc // NS), 8)
    last = n_acc - (NS - 1) * chunk
    mesh = plsc.VectorSubcoreMesh(core_axis_name="c", subcore_axis_name="s")

    @functools.partial(
        pl.kernel,
        mesh=mesh,
        out_type=jax.ShapeDtypeStruct((NC, n_acc, FEAT), jnp.float32),
        scratch_types=[
            pltpu.VMEM((JCHUNK, sub), jnp.int32),
            pltpu.VMEM((JCHUNK, sub), jnp.int32),
            pltpu.VMEM((sub, FEAT), jnp.float32),
            pltpu.VMEM((sub, FEAT), jnp.float32),
            pltpu.VMEM((sub, FEAT), jnp.float32),
            pltpu.VMEM_SHARED((n_acc, FEAT), jnp.float32),
            pltpu.SemaphoreType.DMA,
            pltpu.SemaphoreType.DMA,
        ],
    )
    def segsum(sup_hbm, src_hbm, dst_hbm, zeros_hbm, out_hbm,
               src_v, dst_v, rows_v0, rows_v1, rows_v2, accum, gsem, ssem):
        c = lax.axis_index("c")
        s = lax.axis_index("s")
        wid = s * NC + c
        rows = (rows_v0, rows_v1, rows_v2)

        @pl.when(s < NS - 1)
        def _zero_full():
            pltpu.sync_copy(zeros_hbm, accum.at[pl.ds(s * chunk, chunk)])

        @pl.when(s == NS - 1)
        def _zero_last():
            pltpu.sync_copy(zeros_hbm.at[pl.ds(0, last)],
                            accum.at[pl.ds((NS - 1) * chunk, last)])

        plsc.subcore_barrier()

        def big_body(g, carry):
            pltpu.sync_copy(src_hbm.at[wid, g], src_v)
            pltpu.sync_copy(dst_hbm.at[wid, g], dst_v)
            gh = [None] * JCHUNK
            sh = [None] * JCHUNK
            gh[0] = pltpu.async_copy(sup_hbm.at[src_v.at[0]], rows[0], gsem)
            gh[1] = pltpu.async_copy(sup_hbm.at[src_v.at[1]], rows[1], gsem)
            for j in range(JCHUNK):
                gh[j].wait()
                if j >= 1:
                    sh[j - 1].wait()
                if j + 2 < JCHUNK:
                    gh[j + 2] = pltpu.async_copy(
                        sup_hbm.at[src_v.at[j + 2]], rows[(j + 2) % 3], gsem)
                sh[j] = pltpu.async_copy(
                    rows[j % 3], accum.at[dst_v.at[j]], ssem, add=True)
            sh[JCHUNK - 1].wait()
            return carry

        lax.fori_loop(0, nbig, big_body, 0)
        plsc.subcore_barrier()

        @pl.when(s < NS - 1)
        def _dump_full():
            pltpu.sync_copy(accum.at[pl.ds(s * chunk, chunk)],
                            out_hbm.at[c, pl.ds(s * chunk, chunk)])

        @pl.when(s == NS - 1)
        def _dump_last():
            pltpu.sync_copy(accum.at[pl.ds((NS - 1) * chunk, last)],
                            out_hbm.at[c, pl.ds((NS - 1) * chunk, last)])

    return segsum


_ROWS_BLK = 2000


def _mm2_body(x_ref, w_ref, wl_ref, b_ref, sup_ref, lp_ref):
    xb = x_ref[...]
    sup_ref[...] = jnp.dot(xb, w_ref[...], preferred_element_type=jnp.float32)
    lp_ref[...] = (jnp.dot(xb, wl_ref[...], preferred_element_type=jnp.float32)
                   + b_ref[...])


def _tc_support_loop(x, w, wl, b):
    n = x.shape[0]
    grid = n // _ROWS_BLK
    return pl.pallas_call(
        _mm2_body,
        grid=(grid,),
        in_specs=[
            pl.BlockSpec((_ROWS_BLK, FEAT), lambda i: (i, 0)),
            pl.BlockSpec((FEAT, FEAT), lambda i: (0, 0)),
            pl.BlockSpec((FEAT, FEAT), lambda i: (0, 0)),
            pl.BlockSpec((1, FEAT), lambda i: (0, 0)),
        ],
        out_specs=[
            pl.BlockSpec((_ROWS_BLK, FEAT), lambda i: (i, 0)),
            pl.BlockSpec((_ROWS_BLK, FEAT), lambda i: (i, 0)),
        ],
        out_shape=[
            jax.ShapeDtypeStruct((n, FEAT), jnp.float32),
            jax.ShapeDtypeStruct((n, FEAT), jnp.float32),
        ],
    )(x, w, wl, b.reshape(1, FEAT))


def _relu_mm2_body(p_ref, lp_ref, w_ref, wl_ref, b_ref, sup_ref, lp2_ref):
    h = jnp.maximum(p_ref[0] + p_ref[1] + lp_ref[...], 0.0)
    sup_ref[...] = jnp.dot(h, w_ref[...], preferred_element_type=jnp.float32)
    lp2_ref[...] = (jnp.dot(h, wl_ref[...], preferred_element_type=jnp.float32)
                    + b_ref[...])


def _tc_combine_mm(p, lp, w, wl, b):
    n = lp.shape[0]
    grid = n // _ROWS_BLK
    return pl.pallas_call(
        _relu_mm2_body,
        grid=(grid,),
        in_specs=[
            pl.BlockSpec((2, _ROWS_BLK, FEAT), lambda i: (0, i, 0)),
            pl.BlockSpec((_ROWS_BLK, FEAT), lambda i: (i, 0)),
            pl.BlockSpec((FEAT, FEAT), lambda i: (0, 0)),
            pl.BlockSpec((FEAT, FEAT), lambda i: (0, 0)),
            pl.BlockSpec((1, FEAT), lambda i: (0, 0)),
        ],
        out_specs=[
            pl.BlockSpec((_ROWS_BLK, FEAT), lambda i: (i, 0)),
            pl.BlockSpec((_ROWS_BLK, FEAT), lambda i: (i, 0)),
        ],
        out_shape=[
            jax.ShapeDtypeStruct((n, FEAT), jnp.float32),
            jax.ShapeDtypeStruct((n, FEAT), jnp.float32),
        ],
    )(p, lp, w, wl, b.reshape(1, FEAT))


def _final_body(x_ref, p_ref, lp_ref, o_ref):
    h = jnp.maximum(p_ref[0] + p_ref[1] + lp_ref[...], 0.0)
    o_ref[...] = (x_ref[...] + h) * 0.5


def _tc_final(x, p, lp):
    n = x.shape[0]
    grid = n // _ROWS_BLK
    return pl.pallas_call(
        _final_body,
        grid=(grid,),
        in_specs=[
            pl.BlockSpec((_ROWS_BLK, FEAT), lambda i: (i, 0)),
            pl.BlockSpec((2, _ROWS_BLK, FEAT), lambda i: (0, i, 0)),
            pl.BlockSpec((_ROWS_BLK, FEAT), lambda i: (i, 0)),
        ],
        out_specs=pl.BlockSpec((_ROWS_BLK, FEAT), lambda i: (i, 0)),
        out_shape=jax.ShapeDtypeStruct((n, FEAT), jnp.float32),
    )(x, p, lp)


def kernel(inputs, adj_mat, W1, W1_loop, b1, W2, W2_loop, b2):
    n = inputs.shape[0]
    e = adj_mat.shape[1]
    per_tile = -(-e // NW)
    nbig = -(-per_tile // (JCHUNK * SUB_CAP))
    sub = -(-per_tile // (JCHUNK * nbig))
    epad = NW * nbig * JCHUNK * sub - e
    n_acc = n if epad == 0 else _round_up(n + 128, 128)
    zchunk = _round_up(-(-n_acc // NS), 8)

    src = adj_mat[0]
    dst = adj_mat[1]
    if epad:
        src = jnp.concatenate([src, jnp.zeros((epad,), jnp.int32)])
        dst = jnp.concatenate(
            [dst, n + (jnp.arange(epad, dtype=jnp.int32) % 96)])
    src_r = src.reshape(NW, nbig, JCHUNK, sub)
    dst_r = dst.reshape(NW, nbig, JCHUNK, sub)
    zeros = jnp.zeros((zchunk, FEAT), jnp.float32)

    segsum = _make_segsum(n, nbig, sub, n_acc)

    sup1, lp1 = _tc_support_loop(inputs, W1, W1_loop, b1)
    p1 = segsum(sup1, src_r, dst_r, zeros)
    sup2, lp2 = _tc_combine_mm(p1, lp1, W2, W2_loop, b2)
    p2 = segsum(sup2, src_r, dst_r, zeros)
    return _tc_final(inputs, p2, lp2)


# speedup vs baseline: 11.2447x; 1.0418x over previous
import functools

import jax
import jax.numpy as jnp
from jax import lax
from jax.experimental import pallas as pl
from jax.experimental.pallas import tpu as pltpu
from jax.experimental.pallas import tpu_sc as plsc

FEAT = 128
NC = 2
NS = 16
NW = NC * NS

JCHUNK = 10
SUB_CAP = 52


def _round_up(x, m):
    return (x + m - 1) // m * m


@functools.lru_cache(maxsize=None)
def _make_segsum(n_nodes, nbig, sub, n_acc):
    chunk = _round_up(-(-n_acc // NS), 8)
    last = n_acc - (NS - 1) * chunk
    mesh = plsc.VectorSubcoreMesh(core_axis_name="c", subcore_axis_name="s")

    @functools.partial(
        pl.kernel,
        mesh=mesh,
        out_type=jax.ShapeDtypeStruct((NC, n_acc, FEAT), jnp.float32),
        scratch_types=[
            pltpu.VMEM((2, JCHUNK, sub), jnp.int32),
            pltpu.VMEM((2, JCHUNK, sub), jnp.int32),
            pltpu.VMEM((sub, FEAT), jnp.float32),
            pltpu.VMEM((sub, FEAT), jnp.float32),
            pltpu.VMEM((sub, FEAT), jnp.float32),
            pltpu.VMEM((sub, FEAT), jnp.float32),
            pltpu.VMEM_SHARED((n_acc, FEAT), jnp.float32),
            pltpu.SemaphoreType.DMA,
            pltpu.SemaphoreType.DMA,
            pltpu.SemaphoreType.DMA,
        ],
    )
    def segsum(sup_hbm, src_hbm, dst_hbm, zeros_hbm, out_hbm,
               src_v, dst_v, rows_v0, rows_v1, rows_v2, rows_v3, accum,
               gsem, ssem, isem):
        c = lax.axis_index("c")
        s = lax.axis_index("s")
        wid = s * NC + c
        rows = (rows_v0, rows_v1, rows_v2, rows_v3)

        NBUF = 4
        NOUT = 3

        def chunk_pipeline(sv, dv, prefired):
            gh = [None] * JCHUNK
            sh = [None] * JCHUNK
            for q in range(NOUT):
                if prefired:
                    gh[q] = pltpu.make_async_copy(
                        sup_hbm.at[sv.at[q]], rows[q], gsem)
                else:
                    gh[q] = pltpu.async_copy(
                        sup_hbm.at[sv.at[q]], rows[q], gsem)
            for j in range(JCHUNK):
                gh[j].wait()
                if j >= 1:
                    sh[j - 1].wait()
                if j + NOUT < JCHUNK:
                    gh[j + NOUT] = pltpu.async_copy(
                        sup_hbm.at[sv.at[j + NOUT]],
                        rows[(j + NOUT) % NBUF], gsem)
                sh[j] = pltpu.async_copy(
                    rows[j % NBUF], accum.at[dv.at[j]], ssem, add=True)
            sh[JCHUNK - 1].wait()

        def prefetch_idx(g, slot):
            pltpu.async_copy(src_hbm.at[wid, g], src_v.at[slot], isem)
            pltpu.async_copy(dst_hbm.at[wid, g], dst_v.at[slot], isem)

        def land_idx(g, slot):
            pltpu.make_async_copy(src_hbm.at[wid, g],
                                  src_v.at[slot], isem).wait()
            pltpu.make_async_copy(dst_hbm.at[wid, g],
                                  dst_v.at[slot], isem).wait()

        pltpu.sync_copy(src_hbm.at[wid, 0], src_v.at[0])
        pltpu.sync_copy(dst_hbm.at[wid, 0], dst_v.at[0])
        for _q in range(3):
            pltpu.async_copy(sup_hbm.at[src_v.at[0].at[_q]], rows[_q], gsem)

        @pl.when(s < NS - 1)
        def _zero_full():
            pltpu.sync_copy(zeros_hbm, accum.at[pl.ds(s * chunk, chunk)])

        @pl.when(s == NS - 1)
        def _zero_last():
            pltpu.sync_copy(zeros_hbm.at[pl.ds(0, last)],
                            accum.at[pl.ds((NS - 1) * chunk, last)])

        plsc.subcore_barrier()

        if nbig > 1:
            prefetch_idx(1, 1)
        chunk_pipeline(src_v.at[0], dst_v.at[0], prefired=True)
        if nbig > 1:
            land_idx(1, 1)

        def big_body(g, carry):
            slot = lax.rem(g, 2)
            sv = src_v.at[slot]
            dv = dst_v.at[slot]

            @pl.when(g + 1 < nbig)
            def _prefetch():
                prefetch_idx(g + 1, 1 - slot)

            chunk_pipeline(sv, dv, prefired=False)

            @pl.when(g + 1 < nbig)
            def _land():
                land_idx(g + 1, 1 - slot)
            return carry

        lax.fori_loop(1, nbig, big_body, 0)
        plsc.subcore_barrier()

        @pl.when(s < NS - 1)
        def _dump_full():
            pltpu.sync_copy(accum.at[pl.ds(s * chunk, chunk)],
                            out_hbm.at[c, pl.ds(s * chunk, chunk)])

        @pl.when(s == NS - 1)
        def _dump_last():
            pltpu.sync_copy(accum.at[pl.ds((NS - 1) * chunk, last)],
                            out_hbm.at[c, pl.ds((NS - 1) * chunk, last)])

    return segsum


_ROWS_BLK = 2000


def _lp_body(x_ref, wl_ref, b_ref, lp_ref):
    lp_ref[...] = (jnp.dot(x_ref[...], wl_ref[...],
                           preferred_element_type=jnp.float32) + b_ref[...])


def _tc_loop_mm(x, wl, b):
    n = x.shape[0]
    grid = n // _ROWS_BLK
    return pl.pallas_call(
        _lp_body,
        grid=(grid,),
        in_specs=[
            pl.BlockSpec((_ROWS_BLK, FEAT), lambda i: (i, 0)),
            pl.BlockSpec((FEAT, FEAT), lambda i: (0, 0)),
            pl.BlockSpec((1, FEAT), lambda i: (0, 0)),
        ],
        out_specs=pl.BlockSpec((_ROWS_BLK, FEAT), lambda i: (i, 0)),
        out_shape=jax.ShapeDtypeStruct((n, FEAT), jnp.float32),
    )(x, wl, b.reshape(1, FEAT))


def _h_body(p_ref, lp_ref, w_ref, h_ref):
    agg = p_ref[0] + p_ref[1]
    h_ref[...] = jnp.maximum(
        jnp.dot(agg, w_ref[...], preferred_element_type=jnp.float32)
        + lp_ref[...], 0.0)


def _tc_h(p, lp, w):
    n = lp.shape[0]
    grid = n // _ROWS_BLK
    return pl.pallas_call(
        _h_body,
        grid=(grid,),
        in_specs=[
            pl.BlockSpec((2, _ROWS_BLK, FEAT), lambda i: (0, i, 0)),
            pl.BlockSpec((_ROWS_BLK, FEAT), lambda i: (i, 0)),
            pl.BlockSpec((FEAT, FEAT), lambda i: (0, 0)),
        ],
        out_specs=pl.BlockSpec((_ROWS_BLK, FEAT), lambda i: (i, 0)),
        out_shape=jax.ShapeDtypeStruct((n, FEAT), jnp.float32),
    )(p, lp, w)


def _final_body(x_ref, p_ref, lp_ref, w_ref, o_ref):
    agg = p_ref[0] + p_ref[1]
    h = jnp.maximum(
        jnp.dot(agg, w_ref[...], preferred_element_type=jnp.float32)
        + lp_ref[...], 0.0)
    o_ref[...] = (x_ref[...] + h) * 0.5


def _tc_final(x, p, lp, w):
    n = x.shape[0]
    grid = n // _ROWS_BLK
    return pl.pallas_call(
        _final_body,
        grid=(grid,),
        in_specs=[
            pl.BlockSpec((_ROWS_BLK, FEAT), lambda i: (i, 0)),
            pl.BlockSpec((2, _ROWS_BLK, FEAT), lambda i: (0, i, 0)),
            pl.BlockSpec((_ROWS_BLK, FEAT), lambda i: (i, 0)),
            pl.BlockSpec((FEAT, FEAT), lambda i: (0, 0)),
        ],
        out_specs=pl.BlockSpec((_ROWS_BLK, FEAT), lambda i: (i, 0)),
        out_shape=jax.ShapeDtypeStruct((n, FEAT), jnp.float32),
    )(x, p, lp, w)


def kernel(inputs, adj_mat, W1, W1_loop, b1, W2, W2_loop, b2):
    n = inputs.shape[0]
    e = adj_mat.shape[1]
    per_tile = -(-e // NW)
    nbig = -(-per_tile // (JCHUNK * SUB_CAP))
    sub = -(-per_tile // (JCHUNK * nbig))
    epad = NW * nbig * JCHUNK * sub - e
    n_acc = n if epad == 0 else _round_up(n + 128, 128)
    zchunk = _round_up(-(-n_acc // NS), 8)

    src = adj_mat[0]
    dst = adj_mat[1]
    if epad:
        src = jnp.concatenate([src, jnp.zeros((epad,), jnp.int32)])
        dst = jnp.concatenate(
            [dst, n + (jnp.arange(epad, dtype=jnp.int32) % 96)])
    src_r = src.reshape(NW, nbig, JCHUNK, sub)
    dst_r = dst.reshape(NW, nbig, JCHUNK, sub)
    zeros = jnp.zeros((zchunk, FEAT), jnp.float32)

    segsum = _make_segsum(n, nbig, sub, n_acc)

    p1 = segsum(inputs, src_r, dst_r, zeros)
    lp1 = _tc_loop_mm(inputs, W1_loop, b1)
    h1 = _tc_h(p1, lp1, W1)
    p2 = segsum(h1, src_r, dst_r, zeros)
    lp2 = _tc_loop_mm(h1, W2_loop, b2)
    return _tc_final(inputs, p2, lp2, W2)


# speedup vs baseline: 11.7442x; 1.0444x over previous
import functools

import jax
import jax.numpy as jnp
from jax import lax
from jax.experimental import pallas as pl
from jax.experimental.pallas import tpu as pltpu
from jax.experimental.pallas import tpu_sc as plsc

FEAT = 128
NC = 2
NS = 16
NW = NC * NS

JCHUNK = 10
SUB_CAP = 104


def _round_up(x, m):
    return (x + m - 1) // m * m


@functools.lru_cache(maxsize=None)
def _make_segsum(n_nodes, nbig, sub, n_acc):
    chunk = _round_up(-(-n_acc // NS), 8)
    last = n_acc - (NS - 1) * chunk
    mesh = plsc.VectorSubcoreMesh(core_axis_name="c", subcore_axis_name="s")

    @functools.partial(
        pl.kernel,
        mesh=mesh,
        out_type=jax.ShapeDtypeStruct((NC, n_acc, FEAT), jnp.float32),
        scratch_types=[
            pltpu.VMEM((2, JCHUNK, sub), jnp.int32),
            pltpu.VMEM((2, JCHUNK, sub), jnp.int32),
            pltpu.VMEM((sub, FEAT), jnp.float32),
            pltpu.VMEM((sub, FEAT), jnp.float32),
            pltpu.VMEM((sub, FEAT), jnp.float32),
            pltpu.VMEM_SHARED((n_acc, FEAT), jnp.float32),
            pltpu.SemaphoreType.DMA,
            pltpu.SemaphoreType.DMA,
            pltpu.SemaphoreType.DMA,
        ],
    )
    def segsum(sup_hbm, src_hbm, dst_hbm, zeros_hbm, out_hbm,
               src_v, dst_v, rows_v0, rows_v1, rows_v2, accum,
               gsem, ssem, isem):
        c = lax.axis_index("c")
        s = lax.axis_index("s")
        wid = s * NC + c
        rows = (rows_v0, rows_v1, rows_v2)

        def chunk_pipeline(sv, dv, prefired):
            gh = [None] * JCHUNK
            sh = [None] * JCHUNK
            if prefired:
                gh[0] = pltpu.make_async_copy(
                    sup_hbm.at[sv.at[0]], rows[0], gsem)
                gh[1] = pltpu.make_async_copy(
                    sup_hbm.at[sv.at[1]], rows[1], gsem)
            else:
                gh[0] = pltpu.async_copy(sup_hbm.at[sv.at[0]], rows[0], gsem)
                gh[1] = pltpu.async_copy(sup_hbm.at[sv.at[1]], rows[1], gsem)
            for j in range(JCHUNK):
                gh[j].wait()
                if j >= 1:
                    sh[j - 1].wait()
                if j + 2 < JCHUNK:
                    gh[j + 2] = pltpu.async_copy(
                        sup_hbm.at[sv.at[j + 2]], rows[(j + 2) % 3], gsem)
                sh[j] = pltpu.async_copy(
                    rows[j % 3], accum.at[dv.at[j]], ssem, add=True)
            sh[JCHUNK - 1].wait()

        def prefetch_idx(g, slot):
            pltpu.async_copy(src_hbm.at[wid, g], src_v.at[slot], isem)
            pltpu.async_copy(dst_hbm.at[wid, g], dst_v.at[slot], isem)

        def land_idx(g, slot):
            pltpu.make_async_copy(src_hbm.at[wid, g],
                                  src_v.at[slot], isem).wait()
            pltpu.make_async_copy(dst_hbm.at[wid, g],
                                  dst_v.at[slot], isem).wait()

        pltpu.sync_copy(src_hbm.at[wid, 0], src_v.at[0])
        pltpu.sync_copy(dst_hbm.at[wid, 0], dst_v.at[0])
        pltpu.async_copy(sup_hbm.at[src_v.at[0].at[0]], rows[0], gsem)
        pltpu.async_copy(sup_hbm.at[src_v.at[0].at[1]], rows[1], gsem)

        @pl.when(s < NS - 1)
        def _zero_full():
            pltpu.sync_copy(zeros_hbm, accum.at[pl.ds(s * chunk, chunk)])

        @pl.when(s == NS - 1)
        def _zero_last():
            pltpu.sync_copy(zeros_hbm.at[pl.ds(0, last)],
                            accum.at[pl.ds((NS - 1) * chunk, last)])

        plsc.subcore_barrier()

        if nbig > 1:
            prefetch_idx(1, 1)
        chunk_pipeline(src_v.at[0], dst_v.at[0], prefired=True)
        if nbig > 1:
            land_idx(1, 1)

        def big_body(g, carry):
            slot = lax.rem(g, 2)
            sv = src_v.at[slot]
            dv = dst_v.at[slot]

            @pl.when(g + 1 < nbig)
            def _prefetch():
                prefetch_idx(g + 1, 1 - slot)

            chunk_pipeline(sv, dv, prefired=False)

            @pl.when(g + 1 < nbig)
            def _land():
                land_idx(g + 1, 1 - slot)
            return carry

        lax.fori_loop(1, nbig, big_body, 0)
        plsc.subcore_barrier()

        @pl.when(s < NS - 1)
        def _dump_full():
            pltpu.sync_copy(accum.at[pl.ds(s * chunk, chunk)],
                            out_hbm.at[c, pl.ds(s * chunk, chunk)])

        @pl.when(s == NS - 1)
        def _dump_last():
            pltpu.sync_copy(accum.at[pl.ds((NS - 1) * chunk, last)],
                            out_hbm.at[c, pl.ds((NS - 1) * chunk, last)])

    return segsum


_ROWS_BLK = 2000


def _lp_body(x_ref, wl_ref, b_ref, lp_ref):
    lp_ref[...] = (jnp.dot(x_ref[...], wl_ref[...],
                           preferred_element_type=jnp.float32) + b_ref[...])


def _tc_loop_mm(x, wl, b):
    n = x.shape[0]
    grid = n // _ROWS_BLK
    return pl.pallas_call(
        _lp_body,
        grid=(grid,),
        in_specs=[
            pl.BlockSpec((_ROWS_BLK, FEAT), lambda i: (i, 0)),
            pl.BlockSpec((FEAT, FEAT), lambda i: (0, 0)),
            pl.BlockSpec((1, FEAT), lambda i: (0, 0)),
        ],
        out_specs=pl.BlockSpec((_ROWS_BLK, FEAT), lambda i: (i, 0)),
        out_shape=jax.ShapeDtypeStruct((n, FEAT), jnp.float32),
    )(x, wl, b.reshape(1, FEAT))


def _h_body(p_ref, lp_ref, w_ref, h_ref):
    agg = p_ref[0] + p_ref[1]
    h_ref[...] = jnp.maximum(
        jnp.dot(agg, w_ref[...], preferred_element_type=jnp.float32)
        + lp_ref[...], 0.0)


def _tc_h(p, lp, w):
    n = lp.shape[0]
    grid = n // _ROWS_BLK
    return pl.pallas_call(
        _h_body,
        grid=(grid,),
        in_specs=[
            pl.BlockSpec((2, _ROWS_BLK, FEAT), lambda i: (0, i, 0)),
            pl.BlockSpec((_ROWS_BLK, FEAT), lambda i: (i, 0)),
            pl.BlockSpec((FEAT, FEAT), lambda i: (0, 0)),
        ],
        out_specs=pl.BlockSpec((_ROWS_BLK, FEAT), lambda i: (i, 0)),
        out_shape=jax.ShapeDtypeStruct((n, FEAT), jnp.float32),
    )(p, lp, w)


def _final_body(x_ref, p_ref, lp_ref, w_ref, o_ref):
    agg = p_ref[0] + p_ref[1]
    h = jnp.maximum(
        jnp.dot(agg, w_ref[...], preferred_element_type=jnp.float32)
        + lp_ref[...], 0.0)
    o_ref[...] = (x_ref[...] + h) * 0.5


def _tc_final(x, p, lp, w):
    n = x.shape[0]
    grid = n // _ROWS_BLK
    return pl.pallas_call(
        _final_body,
        grid=(grid,),
        in_specs=[
            pl.BlockSpec((_ROWS_BLK, FEAT), lambda i: (i, 0)),
            pl.BlockSpec((2, _ROWS_BLK, FEAT), lambda i: (0, i, 0)),
            pl.BlockSpec((_ROWS_BLK, FEAT), lambda i: (i, 0)),
            pl.BlockSpec((FEAT, FEAT), lambda i: (0, 0)),
        ],
        out_specs=pl.BlockSpec((_ROWS_BLK, FEAT), lambda i: (i, 0)),
        out_shape=jax.ShapeDtypeStruct((n, FEAT), jnp.float32),
    )(x, p, lp, w)


def kernel(inputs, adj_mat, W1, W1_loop, b1, W2, W2_loop, b2):
    n = inputs.shape[0]
    e = adj_mat.shape[1]
    per_tile = -(-e // NW)
    nbig = -(-per_tile // (JCHUNK * SUB_CAP))
    sub = -(-per_tile // (JCHUNK * nbig))
    epad = NW * nbig * JCHUNK * sub - e
    n_acc = n if epad == 0 else _round_up(n + 128, 128)
    zchunk = _round_up(-(-n_acc // NS), 8)

    src = adj_mat[0]
    dst = adj_mat[1]
    if epad:
        src = jnp.concatenate([src, jnp.zeros((epad,), jnp.int32)])
        dst = jnp.concatenate(
            [dst, n + (jnp.arange(epad, dtype=jnp.int32) % 96)])
    src_r = src.reshape(NW, nbig, JCHUNK, sub)
    dst_r = dst.reshape(NW, nbig, JCHUNK, sub)
    zeros = jnp.zeros((zchunk, FEAT), jnp.float32)

    segsum = _make_segsum(n, nbig, sub, n_acc)

    p1 = segsum(inputs, src_r, dst_r, zeros)
    lp1 = _tc_loop_mm(inputs, W1_loop, b1)
    h1 = _tc_h(p1, lp1, W1)
    p2 = segsum(h1, src_r, dst_r, zeros)
    lp2 = _tc_loop_mm(h1, W2_loop, b2)
    return _tc_final(inputs, p2, lp2, W2)
